# baseline (speedup 1.0000x reference)
import functools

import jax
import jax.numpy as jnp
from jax import lax
from jax.experimental import pallas as pl
from jax.experimental.pallas import tpu as pltpu
from jax.experimental.pallas import tpu_sc as plsc

_LANES = 16
_CHUNK = 128


def _scale_block(dose, beta_g, bias_g):
    s = dose / (dose + 2.0)
    s2 = s * s
    l1p = 2.0 * s * (1.0 + s2 * (1.0 / 3.0 + s2 * (1.0 / 5.0
                     + s2 * (1.0 / 7.0 + s2 * (1.0 / 9.0)))))
    y = l1p * beta_g + bias_g
    sig_y = 1.0 / (1.0 + jnp.exp(-y))
    sig_b = 1.0 / (1.0 + jnp.exp(-bias_g))
    return sig_y - sig_b


def kernel(drugs, doses, embedding, beta, bias):
    B = drugs.shape[0]
    V, D = embedding.shape
    drugs = drugs.reshape(-1).astype(jnp.int32)
    doses = doses.reshape(-1).astype(jnp.float32)

    info = plsc.get_sparse_core_info()
    NC, NS = info.num_cores, info.num_subcores
    NW = NC * NS
    b_per_w = B // NW
    n_chunks = b_per_w // _CHUNK
    drugs2 = drugs.reshape(NW * n_chunks, _CHUNK)

    mesh = plsc.VectorSubcoreMesh(core_axis_name="c", subcore_axis_name="s")

    @functools.partial(
        pl.kernel,
        mesh=mesh,
        compiler_params=pltpu.CompilerParams(use_tc_tiling_on_sc=False),
        out_type=jax.ShapeDtypeStruct((B, D), jnp.float32),
        scratch_types=[
            pltpu.VMEM((n_chunks, _CHUNK), jnp.int32),
            pltpu.VMEM((b_per_w,), jnp.float32),
            pltpu.VMEM((b_per_w,), jnp.float32),
            pltpu.VMEM((b_per_w,), jnp.float32),
            pltpu.VMEM((b_per_w,), jnp.float32),
            pltpu.VMEM((b_per_w, D), jnp.float32),
            pltpu.SemaphoreType.DMA,
        ],
    )
    def _k(drugs_hbm, doses_hbm, emb_hbm, beta_hbm, bias_hbm, out_hbm,
           idx_v, dose_v, betag_v, biasg_v, scale_v, rows_v, sem):
        wid = lax.axis_index("s") * NC + lax.axis_index("c")
        base = wid * b_per_w
        pltpu.sync_copy(drugs_hbm.at[pl.ds(wid * n_chunks, n_chunks)], idx_v)
        pltpu.sync_copy(doses_hbm.at[pl.ds(base, b_per_w)], dose_v)

        copies = []
        for j in range(n_chunks):
            idx_row = idx_v.at[j]
            sl = pl.ds(j * _CHUNK, _CHUNK)
            copies.append(pltpu.async_copy(emb_hbm.at[idx_row], rows_v.at[sl], sem))
            copies.append(pltpu.async_copy(beta_hbm.at[idx_row], betag_v.at[sl], sem))
            copies.append(pltpu.async_copy(bias_hbm.at[idx_row], biasg_v.at[sl], sem))
        for c in copies:
            c.wait()

        def scale_body(g, carry):
            sl = pl.ds(g * _LANES, _LANES)
            scale_v[sl] = _scale_block(dose_v[sl], betag_v[sl], biasg_v[sl])
            return carry
        lax.fori_loop(0, b_per_w // _LANES, scale_body, 0)

        def row_body(g, carry):
            sc_vec = scale_v[pl.ds(g * _LANES, _LANES)]
            for j in range(_LANES):
                splat = lax.broadcast_in_dim(sc_vec[j], (_LANES,), ())
                r = g * _LANES + j
                for k in range(D // _LANES):
                    sl = pl.ds(k * _LANES, _LANES)
                    rows_v[r, sl] = rows_v[r, sl] * splat
            return carry
        lax.fori_loop(0, b_per_w // _LANES, row_body, 0)

        pltpu.sync_copy(rows_v, out_hbm.at[pl.ds(base, b_per_w)])

    return _k(drugs2, doses, embedding, beta, bias)


# speedup vs baseline: 1.4739x; 1.4739x over previous
import functools

import jax
import jax.numpy as jnp
from jax import lax
from jax.experimental import pallas as pl
from jax.experimental.pallas import tpu as pltpu
from jax.experimental.pallas import tpu_sc as plsc

_LANES = 16
_CHUNK = 128


def _scale_block(dose, beta_g, bias_g):
    s = dose / (dose + 2.0)
    s2 = s * s
    l1p = 2.0 * s * (1.0 + s2 * (1.0 / 3.0 + s2 * (1.0 / 5.0
                     + s2 * (1.0 / 7.0 + s2 * (1.0 / 9.0)))))
    y = l1p * beta_g + bias_g
    sig_y = 1.0 / (1.0 + jnp.exp(-y))
    sig_b = 1.0 / (1.0 + jnp.exp(-bias_g))
    return sig_y - sig_b


def kernel(drugs, doses, embedding, beta, bias):
    B = drugs.shape[0]
    V, D = embedding.shape
    drugs = drugs.reshape(-1).astype(jnp.int32)
    doses = doses.reshape(-1).astype(jnp.float32)

    info = plsc.get_sparse_core_info()
    NC, NS = info.num_cores, info.num_subcores
    NW = NC * NS
    b_per_w = B // NW
    n_chunks = b_per_w // _CHUNK

    mesh = plsc.VectorSubcoreMesh(core_axis_name="c", subcore_axis_name="s")

    @functools.partial(
        pl.kernel,
        mesh=mesh,
        out_type=jax.ShapeDtypeStruct((B, D), jnp.float32),
        scratch_types=[
            pltpu.VMEM((n_chunks, _CHUNK), jnp.int32),
            pltpu.VMEM((b_per_w,), jnp.float32),
            pltpu.VMEM((b_per_w,), jnp.float32),
            pltpu.VMEM((b_per_w,), jnp.float32),
            pltpu.VMEM((b_per_w,), jnp.float32),
            pltpu.VMEM((b_per_w, D), jnp.float32),
            pltpu.SemaphoreType.DMA,
            pltpu.SemaphoreType.DMA,
        ],
    )
    def _k(drugs_hbm, doses_hbm, emb_hbm, beta_hbm, bias_hbm, out_hbm,
           idx_v, dose_v, betag_v, biasg_v, scale_v, rows_v, sem, rsem):
        wid = lax.axis_index("s") * NC + lax.axis_index("c")
        base = wid * b_per_w
        for j in range(n_chunks):
            pltpu.sync_copy(drugs_hbm.at[pl.ds(base + j * _CHUNK, _CHUNK)],
                            idx_v.at[j])
        pltpu.sync_copy(doses_hbm.at[pl.ds(base, b_per_w)], dose_v)

        copies = []
        for j in range(n_chunks):
            idx_row = idx_v.at[j]
            sl = pl.ds(j * _CHUNK, _CHUNK)
            copies.append(pltpu.async_copy(beta_hbm.at[idx_row], betag_v.at[sl], sem))
            copies.append(pltpu.async_copy(bias_hbm.at[idx_row], biasg_v.at[sl], sem))

        def row_dma_body(g, carry):
            iv = idx_v[g // (_CHUNK // _LANES),
                       pl.ds((g % (_CHUNK // _LANES)) * _LANES, _LANES)]
            for l in range(_LANES):
                r = g * _LANES + l
                pltpu.async_copy(emb_hbm.at[iv[l]], rows_v.at[r], rsem)
            return carry
        lax.fori_loop(0, b_per_w // _LANES, row_dma_body, 0)

        def scale_body(g, carry):
            sl = pl.ds(g * _LANES, _LANES)
            scale_v[sl] = _scale_block(dose_v[sl], betag_v[sl], biasg_v[sl])
            return carry
        for c in copies:
            c.wait()
        lax.fori_loop(0, b_per_w // _LANES, scale_body, 0)

        pltpu.make_async_copy(emb_hbm.at[pl.ds(0, b_per_w)], rows_v, rsem).wait()

        def row_body(g, carry):
            sc_vec = scale_v[pl.ds(g * _LANES, _LANES)]
            for j in range(_LANES):
                splat = lax.broadcast_in_dim(sc_vec[j], (_LANES,), ())
                r = g * _LANES + j
                for k in range(D // _LANES):
                    sl = pl.ds(k * _LANES, _LANES)
                    rows_v[r, sl] = rows_v[r, sl] * splat
            return carry
        lax.fori_loop(0, b_per_w // _LANES, row_body, 0)

        pltpu.sync_copy(rows_v, out_hbm.at[pl.ds(base, b_per_w)])

    return _k(drugs, doses, embedding, beta, bias)


# speedup vs baseline: 1.6577x; 1.1247x over previous
import functools

import jax
import jax.numpy as jnp
from jax import lax
from jax.experimental import pallas as pl
from jax.experimental.pallas import tpu as pltpu
from jax.experimental.pallas import tpu_sc as plsc

_LANES = 16
_ICHUNK = 128
_CHUNK = 2048


def _scale_block(dose, beta_g, bias_g):
    s = dose / (dose + 2.0)
    s2 = s * s
    l1p = 2.0 * s * (1.0 + s2 * (1.0 / 3.0 + s2 * (1.0 / 5.0
                     + s2 * (1.0 / 7.0 + s2 * (1.0 / 9.0)))))
    y = l1p * beta_g + bias_g
    sig_y = 1.0 / (1.0 + jnp.exp(-y))
    sig_b = 1.0 / (1.0 + jnp.exp(-bias_g))
    return sig_y - sig_b


def kernel(drugs, doses, embedding, beta, bias):
    B = drugs.shape[0]
    V, D = embedding.shape
    drugs = drugs.reshape(-1).astype(jnp.int32)
    doses = doses.reshape(-1).astype(jnp.float32)
    emb_t = embedding.T

    info = plsc.get_sparse_core_info()
    NC, NS = info.num_cores, info.num_subcores
    NW = NC * NS
    pa = B // NS
    rows_per_w = D // NW

    mesh = plsc.VectorSubcoreMesh(core_axis_name="c", subcore_axis_name="s")

    @functools.partial(
        pl.kernel,
        mesh=mesh,
        compiler_params=pltpu.CompilerParams(needs_layout_passes=False),
        out_type=jax.ShapeDtypeStruct((D, B), jnp.float32),
        scratch_types=[
            pltpu.VMEM((B,), jnp.int32),
            pltpu.VMEM((pa,), jnp.float32),
            pltpu.VMEM((pa,), jnp.float32),
            pltpu.VMEM((pa,), jnp.float32),
            pltpu.VMEM((pa,), jnp.float32),
            pltpu.VMEM((V,), jnp.float32),
            pltpu.VMEM((_CHUNK,), jnp.float32),
            pltpu.VMEM((_CHUNK,), jnp.float32),
            pltpu.VMEM_SHARED((B,), jnp.float32),
            pltpu.SemaphoreType.DMA,
        ],
    )
    def _k(drugs_hbm, doses_hbm, embt_hbm, beta_hbm, bias_hbm, out_hbm,
           idx_v, dose_v, betag_v, biasg_v, scale_a, row_v, scale_c, out_c,
           scale_sh, sem):
        sid = lax.axis_index("s")
        cid = lax.axis_index("c")
        wid = cid * NS + sid
        a0 = sid * pa

        pltpu.sync_copy(drugs_hbm, idx_v)
        pltpu.sync_copy(doses_hbm.at[pl.ds(a0, pa)], dose_v)

        copies = []
        for j in range(pa // _ICHUNK):
            isl = idx_v.at[pl.ds(a0 + j * _ICHUNK, _ICHUNK)]
            sl = pl.ds(j * _ICHUNK, _ICHUNK)
            copies.append(pltpu.async_copy(beta_hbm.at[isl], betag_v.at[sl], sem))
            copies.append(pltpu.async_copy(bias_hbm.at[isl], biasg_v.at[sl], sem))
        for c in copies:
            c.wait()

        def sbody(g, carry):
            sl = pl.ds(g * _LANES, _LANES)
            scale_a[sl] = _scale_block(dose_v[sl], betag_v[sl], biasg_v[sl])
            return carry
        lax.fori_loop(0, pa // _LANES, sbody, 0)
        pltpu.sync_copy(scale_a, scale_sh.at[pl.ds(a0, pa)])
        plsc.subcore_barrier()

        for rr in range(rows_per_w):
            r = wid * rows_per_w + rr
            pltpu.sync_copy(embt_hbm.at[r], row_v)
            for k in range(B // _CHUNK):
                pltpu.sync_copy(scale_sh.at[pl.ds(k * _CHUNK, _CHUNK)], scale_c)

                def gbody(v, carry):
                    sl = pl.ds(v * _LANES, _LANES)
                    iv = idx_v[pl.ds(k * _CHUNK + v * _LANES, _LANES)]
                    out_c[sl] = plsc.load_gather(row_v, [iv]) * scale_c[sl]
                    return carry
                lax.fori_loop(0, _CHUNK // _LANES, gbody, 0)
                pltpu.sync_copy(out_c, out_hbm.at[r, pl.ds(k * _CHUNK, _CHUNK)])

    return _k(drugs, doses, emb_t, beta, bias).T


# speedup vs baseline: 2.1714x; 1.3098x over previous
import functools

import jax
import jax.numpy as jnp
from jax import lax
from jax.experimental import pallas as pl
from jax.experimental.pallas import tpu as pltpu
from jax.experimental.pallas import tpu_sc as plsc

_LANES = 16
_ICHUNK = 128
_CHUNK = 2048


def _scale_block(dose, beta_g, bias_g):
    s = dose / (dose + 2.0)
    s2 = s * s
    l1p = 2.0 * s * (1.0 + s2 * (1.0 / 3.0 + s2 * (1.0 / 5.0
                     + s2 * (1.0 / 7.0 + s2 * (1.0 / 9.0)))))
    y = l1p * beta_g + bias_g
    sig_y = 1.0 / (1.0 + jnp.exp(-y))
    sig_b = 1.0 / (1.0 + jnp.exp(-bias_g))
    return sig_y - sig_b


def kernel(drugs, doses, embedding, beta, bias):
    B = drugs.shape[0]
    V, D = embedding.shape
    drugs = drugs.reshape(-1).astype(jnp.int32)
    doses = doses.reshape(-1).astype(jnp.float32)
    emb_t = embedding.T

    info = plsc.get_sparse_core_info()
    NC, NS = info.num_cores, info.num_subcores
    NW = NC * NS
    pa = B // NS
    rows_per_w = D // NW

    mesh = plsc.VectorSubcoreMesh(core_axis_name="c", subcore_axis_name="s")

    @functools.partial(
        pl.kernel,
        mesh=mesh,
        compiler_params=pltpu.CompilerParams(needs_layout_passes=False),
        out_type=jax.ShapeDtypeStruct((D, B), jnp.float32),
        scratch_types=[
            pltpu.VMEM((B,), jnp.int32),
            pltpu.VMEM((pa,), jnp.float32),
            pltpu.VMEM((pa,), jnp.float32),
            pltpu.VMEM((pa,), jnp.float32),
            pltpu.VMEM((pa,), jnp.float32),
            pltpu.VMEM((V,), jnp.float32),
            pltpu.VMEM((_CHUNK,), jnp.float32),
            pltpu.VMEM((_CHUNK,), jnp.float32),
            pltpu.VMEM_SHARED((B,), jnp.float32),
            pltpu.SemaphoreType.DMA,
        ],
    )
    def _k(drugs_hbm, doses_hbm, embt_hbm, beta_hbm, bias_hbm, out_hbm,
           idx_v, dose_v, betag_v, biasg_v, scale_a, row_v, scale_c, out_c,
           scale_sh, sem):
        sid = lax.axis_index("s")
        cid = lax.axis_index("c")
        wid = cid * NS + sid
        a0 = sid * pa

        pltpu.sync_copy(drugs_hbm, idx_v)
        pltpu.sync_copy(doses_hbm.at[pl.ds(a0, pa)], dose_v)

        copies = []
        for j in range(pa // _ICHUNK):
            isl = idx_v.at[pl.ds(a0 + j * _ICHUNK, _ICHUNK)]
            sl = pl.ds(j * _ICHUNK, _ICHUNK)
            copies.append(pltpu.async_copy(beta_hbm.at[isl], betag_v.at[sl], sem))
            copies.append(pltpu.async_copy(bias_hbm.at[isl], biasg_v.at[sl], sem))
        for c in copies:
            c.wait()

        def sbody(g, carry):
            sl = pl.ds(g * _LANES, _LANES)
            scale_a[sl] = _scale_block(dose_v[sl], betag_v[sl], biasg_v[sl])
            return carry
        lax.fori_loop(0, pa // _LANES, sbody, 0)
        pltpu.sync_copy(scale_a, scale_sh.at[pl.ds(a0, pa)])
        plsc.subcore_barrier()

        for rr in range(rows_per_w):
            r = wid * rows_per_w + rr
            pltpu.sync_copy(embt_hbm.at[r], row_v)
            for k in range(B // _CHUNK):
                pltpu.sync_copy(scale_sh.at[pl.ds(k * _CHUNK, _CHUNK)], scale_c)

                @plsc.parallel_loop(0, _CHUNK // _LANES, unroll=8)
                def gbody(v):
                    sl = pl.ds(v * _LANES, _LANES)
                    iv = idx_v[pl.ds(k * _CHUNK + v * _LANES, _LANES)]
                    out_c[sl] = plsc.load_gather(row_v, [iv]) * scale_c[sl]
                pltpu.sync_copy(out_c, out_hbm.at[r, pl.ds(k * _CHUNK, _CHUNK)])

    return _k(drugs, doses, emb_t, beta, bias).T


# speedup vs baseline: 2.2703x; 1.0456x over previous
import functools

import jax
import jax.numpy as jnp
from jax import lax
from jax.experimental import pallas as pl
from jax.experimental.pallas import tpu as pltpu
from jax.experimental.pallas import tpu_sc as plsc

_LANES = 16
_ICHUNK = 128
_CHUNK = 2048


def _scale_block(dose, beta_g, bias_g):
    s = dose / (dose + 2.0)
    s2 = s * s
    l1p = 2.0 * s * (1.0 + s2 * (1.0 / 3.0 + s2 * (1.0 / 5.0
                     + s2 * (1.0 / 7.0 + s2 * (1.0 / 9.0)))))
    y = l1p * beta_g + bias_g
    sig_y = 1.0 / (1.0 + jnp.exp(-y))
    sig_b = 1.0 / (1.0 + jnp.exp(-bias_g))
    return sig_y - sig_b


def kernel(drugs, doses, embedding, beta, bias):
    B = drugs.shape[0]
    V, D = embedding.shape
    drugs = drugs.reshape(-1).astype(jnp.int32)
    doses = doses.reshape(-1).astype(jnp.float32)
    emb_t = embedding.T

    info = plsc.get_sparse_core_info()
    NC, NS = info.num_cores, info.num_subcores
    NW = NC * NS
    pa = B // NS
    rows_per_w = D // NW

    mesh = plsc.VectorSubcoreMesh(core_axis_name="c", subcore_axis_name="s")

    @functools.partial(
        pl.kernel,
        mesh=mesh,
        compiler_params=pltpu.CompilerParams(needs_layout_passes=False),
        out_type=jax.ShapeDtypeStruct((D, B), jnp.float32),
        scratch_types=[
            pltpu.VMEM((B,), jnp.int32),
            pltpu.VMEM((pa,), jnp.float32),
            pltpu.VMEM((pa,), jnp.float32),
            pltpu.VMEM((pa,), jnp.float32),
            pltpu.VMEM((pa,), jnp.float32),
            pltpu.VMEM((V,), jnp.float32),
            pltpu.VMEM((_CHUNK,), jnp.float32),
            pltpu.VMEM((_CHUNK,), jnp.float32),
            pltpu.VMEM((_CHUNK,), jnp.float32),
            pltpu.VMEM_SHARED((B,), jnp.float32),
            pltpu.SemaphoreType.DMA,
            pltpu.SemaphoreType.DMA,
            pltpu.SemaphoreType.DMA,
        ],
    )
    def _k(drugs_hbm, doses_hbm, embt_hbm, beta_hbm, bias_hbm, out_hbm,
           idx_v, dose_v, betag_v, biasg_v, scale_a, row_v, scale_c, out_c0,
           out_c1, scale_sh, sem, rsem, osem):
        sid = lax.axis_index("s")
        cid = lax.axis_index("c")
        wid = cid * NS + sid
        a0 = sid * pa

        pltpu.sync_copy(drugs_hbm, idx_v)
        row_cp = pltpu.async_copy(embt_hbm.at[wid * rows_per_w], row_v, rsem)
        pltpu.sync_copy(doses_hbm.at[pl.ds(a0, pa)], dose_v)

        copies = []
        for j in range(pa // _ICHUNK):
            isl = idx_v.at[pl.ds(a0 + j * _ICHUNK, _ICHUNK)]
            sl = pl.ds(j * _ICHUNK, _ICHUNK)
            copies.append(pltpu.async_copy(beta_hbm.at[isl], betag_v.at[sl], sem))
            copies.append(pltpu.async_copy(bias_hbm.at[isl], biasg_v.at[sl], sem))
        for c in copies:
            c.wait()

        @plsc.parallel_loop(0, pa // _LANES, unroll=4)
        def sbody(g):
            sl = pl.ds(g * _LANES, _LANES)
            scale_a[sl] = _scale_block(dose_v[sl], betag_v[sl], biasg_v[sl])
        pltpu.sync_copy(scale_a, scale_sh.at[pl.ds(a0, pa)])
        plsc.subcore_barrier()

        out_cps = [None, None]
        for rr in range(rows_per_w):
            r = wid * rows_per_w + rr
            row_cp.wait()
            for k in range(B // _CHUNK):
                pltpu.sync_copy(scale_sh.at[pl.ds(k * _CHUNK, _CHUNK)], scale_c)
                bsel = k % 2
                if out_cps[bsel] is not None:
                    out_cps[bsel].wait()
                ob = out_c0 if bsel == 0 else out_c1

                @plsc.parallel_loop(0, _CHUNK // _LANES, unroll=16)
                def gbody(v):
                    sl = pl.ds(v * _LANES, _LANES)
                    iv = idx_v[pl.ds(k * _CHUNK + v * _LANES, _LANES)]
                    ob[sl] = plsc.load_gather(row_v, [iv]) * scale_c[sl]
                out_cps[bsel] = pltpu.async_copy(
                    ob, out_hbm.at[r, pl.ds(k * _CHUNK, _CHUNK)], osem)
            if rr + 1 < rows_per_w:
                row_cp = pltpu.async_copy(embt_hbm.at[r + 1], row_v, rsem)
        for cp in out_cps:
            if cp is not None:
                cp.wait()

    return _k(drugs, doses, emb_t, beta, bias).T


# speedup vs baseline: 2.4344x; 1.0723x over previous
import functools

import jax
import jax.numpy as jnp
from jax import lax
from jax.experimental import pallas as pl
from jax.experimental.pallas import tpu as pltpu
from jax.experimental.pallas import tpu_sc as plsc

_LANES = 16
_ICHUNK = 128
_CHUNK = 2048


def _scale_block(dose, beta_g, bias_g):
    s = dose / (dose + 2.0)
    s2 = s * s
    l1p = 2.0 * s * (1.0 + s2 * (1.0 / 3.0 + s2 * (1.0 / 5.0
                     + s2 * (1.0 / 7.0 + s2 * (1.0 / 9.0)))))
    y = l1p * beta_g + bias_g
    sig_y = 1.0 / (1.0 + jnp.exp(-y))
    sig_b = 1.0 / (1.0 + jnp.exp(-bias_g))
    return sig_y - sig_b


def kernel(drugs, doses, embedding, beta, bias):
    B = drugs.shape[0]
    V, D = embedding.shape
    drugs = drugs.reshape(-1).astype(jnp.int32)
    doses = doses.reshape(-1).astype(jnp.float32)
    emb_t = embedding.T

    info = plsc.get_sparse_core_info()
    NC, NS = info.num_cores, info.num_subcores
    NW = NC * NS
    pa = B // NS
    rows_per_w = D // NW

    mesh = plsc.VectorSubcoreMesh(core_axis_name="c", subcore_axis_name="s")

    @functools.partial(
        pl.kernel,
        mesh=mesh,
        compiler_params=pltpu.CompilerParams(needs_layout_passes=False),
        out_type=jax.ShapeDtypeStruct((D, B), jnp.float32),
        scratch_types=[
            pltpu.VMEM((B,), jnp.int32),
            pltpu.VMEM((pa,), jnp.float32),
            pltpu.VMEM((pa,), jnp.float32),
            pltpu.VMEM((pa,), jnp.float32),
            pltpu.VMEM((pa,), jnp.float32),
            pltpu.VMEM((V,), jnp.float32),
            pltpu.VMEM((_CHUNK,), jnp.float32),
            pltpu.VMEM((_CHUNK,), jnp.float32),
            pltpu.VMEM((_CHUNK,), jnp.float32),
            pltpu.VMEM_SHARED((B,), jnp.float32),
            pltpu.SemaphoreType.DMA,
            pltpu.SemaphoreType.DMA,
            pltpu.SemaphoreType.DMA,
        ],
    )
    def _k(drugs_hbm, doses_hbm, embt_hbm, beta_hbm, bias_hbm, out_hbm,
           idx_v, dose_v, betag_v, biasg_v, scale_a, row_v, scale_c, out_c0,
           out_c1, scale_sh, sem, rsem, osem):
        sid = lax.axis_index("s")
        cid = lax.axis_index("c")
        wid = cid * NS + sid
        a0 = sid * pa

        pltpu.sync_copy(drugs_hbm, idx_v)
        row_cp = pltpu.async_copy(embt_hbm.at[wid * rows_per_w], row_v, rsem)
        pltpu.sync_copy(doses_hbm.at[pl.ds(a0, pa)], dose_v)

        copies = []
        for j in range(pa // _ICHUNK):
            isl = idx_v.at[pl.ds(a0 + j * _ICHUNK, _ICHUNK)]
            sl = pl.ds(j * _ICHUNK, _ICHUNK)
            copies.append(pltpu.async_copy(beta_hbm.at[isl], betag_v.at[sl], sem))
            copies.append(pltpu.async_copy(bias_hbm.at[isl], biasg_v.at[sl], sem))
        for c in copies:
            c.wait()

        @plsc.parallel_loop(0, pa // _LANES, unroll=4)
        def sbody(g):
            sl = pl.ds(g * _LANES, _LANES)
            scale_a[sl] = _scale_block(dose_v[sl], betag_v[sl], biasg_v[sl])
        pltpu.sync_copy(scale_a, scale_sh.at[pl.ds(a0, pa)])
        plsc.subcore_barrier()

        n_chunks = B // _CHUNK
        n_writes = 0
        for rr in range(rows_per_w):
            r = wid * rows_per_w + rr
            row_cp.wait()

            def pair_body(k2, carry):
                for b, ob in ((0, out_c0), (1, out_c1)):
                    k = k2 * 2 + b
                    pltpu.sync_copy(scale_sh.at[pl.ds(k * _CHUNK, _CHUNK)],
                                    scale_c)

                    @pl.when(carry + b >= 2)
                    def _():
                        pltpu.make_async_copy(
                            out_hbm.at[r, pl.ds(0, _CHUNK)], ob, osem).wait()

                    @plsc.parallel_loop(0, _CHUNK // _LANES, unroll=16)
                    def gbody(v):
                        sl = pl.ds(v * _LANES, _LANES)
                        iv = idx_v[pl.ds(k * _CHUNK + v * _LANES, _LANES)]
                        ob[sl] = plsc.load_gather(row_v, [iv]) * scale_c[sl]
                    pltpu.async_copy(
                        ob, out_hbm.at[r, pl.ds(k * _CHUNK, _CHUNK)], osem)
                return carry + 2
            n_writes = lax.fori_loop(0, n_chunks // 2, pair_body,
                                     jnp.int32(n_writes))
            if rr + 1 < rows_per_w:
                row_cp = pltpu.async_copy(embt_hbm.at[r + 1], row_v, rsem)
        for ob in (out_c0, out_c1):
            pltpu.make_async_copy(out_hbm.at[0, pl.ds(0, _CHUNK)],
                                  ob, osem).wait()

    return _k(drugs, doses, emb_t, beta, bias).T


# speedup vs baseline: 2.4799x; 1.0187x over previous
import functools

import jax
import jax.numpy as jnp
from jax import lax
from jax.experimental import pallas as pl
from jax.experimental.pallas import tpu as pltpu
from jax.experimental.pallas import tpu_sc as plsc

_LANES = 16
_ICHUNK = 128
_CHUNK = 2048


def _scale_block(dose, beta_g, bias_g):
    s = dose / (dose + 2.0)
    s2 = s * s
    l1p = 2.0 * s * (1.0 + s2 * (1.0 / 3.0 + s2 * (1.0 / 5.0
                     + s2 * (1.0 / 7.0 + s2 * (1.0 / 9.0)))))
    y = l1p * beta_g + bias_g
    sig_y = 1.0 / (1.0 + jnp.exp(-y))
    sig_b = 1.0 / (1.0 + jnp.exp(-bias_g))
    return sig_y - sig_b


def kernel(drugs, doses, embedding, beta, bias):
    B = drugs.shape[0]
    V, D = embedding.shape
    drugs = drugs.reshape(-1).astype(jnp.int32)
    doses = doses.reshape(-1).astype(jnp.float32)
    emb_t = embedding.T

    info = plsc.get_sparse_core_info()
    NC, NS = info.num_cores, info.num_subcores
    NW = NC * NS
    pa = B // NS
    rows_per_w = D // NW

    mesh = plsc.VectorSubcoreMesh(core_axis_name="c", subcore_axis_name="s")

    @functools.partial(
        pl.kernel,
        mesh=mesh,
        compiler_params=pltpu.CompilerParams(needs_layout_passes=False),
        out_type=jax.ShapeDtypeStruct((D, B), jnp.float32),
        scratch_types=[
            pltpu.VMEM((B,), jnp.int32),
            pltpu.VMEM((pa,), jnp.int32),
            pltpu.VMEM((pa,), jnp.float32),
            pltpu.VMEM((pa,), jnp.float32),
            pltpu.VMEM((pa,), jnp.float32),
            pltpu.VMEM((pa,), jnp.float32),
            pltpu.VMEM((V,), jnp.float32),
            pltpu.VMEM((_CHUNK,), jnp.float32),
            pltpu.VMEM((_CHUNK,), jnp.float32),
            pltpu.VMEM((_CHUNK,), jnp.float32),
            pltpu.VMEM_SHARED((B,), jnp.float32),
            pltpu.SemaphoreType.DMA,
            pltpu.SemaphoreType.DMA,
            pltpu.SemaphoreType.DMA,
        ],
    )
    def _k(drugs_hbm, doses_hbm, embt_hbm, beta_hbm, bias_hbm, out_hbm,
           idx_v, idx_a, dose_v, betag_v, biasg_v, scale_a, row_v, scale_c,
           out_c0, out_c1, scale_sh, sem, rsem, osem):
        sid = lax.axis_index("s")
        cid = lax.axis_index("c")
        wid = cid * NS + sid
        a0 = sid * pa

        with jax.named_scope("stage_idx"):
            pltpu.sync_copy(drugs_hbm.at[pl.ds(a0, pa)], idx_a)
        row_cp = pltpu.async_copy(embt_hbm.at[wid * rows_per_w], row_v, rsem)
        idx_cp = pltpu.async_copy(drugs_hbm, idx_v, rsem)
        pltpu.sync_copy(doses_hbm.at[pl.ds(a0, pa)], dose_v)

        copies = []
        for j in range(pa // _ICHUNK):
            isl = idx_a.at[pl.ds(j * _ICHUNK, _ICHUNK)]
            sl = pl.ds(j * _ICHUNK, _ICHUNK)
            copies.append(pltpu.async_copy(beta_hbm.at[isl], betag_v.at[sl], sem))
            copies.append(pltpu.async_copy(bias_hbm.at[isl], biasg_v.at[sl], sem))
        with jax.named_scope("phA_gather_wait"):
            for c in copies:
                c.wait()

        @plsc.parallel_loop(0, pa // _LANES, unroll=4)
        def sbody(g):
            sl = pl.ds(g * _LANES, _LANES)
            scale_a[sl] = _scale_block(dose_v[sl], betag_v[sl], biasg_v[sl])
        pltpu.sync_copy(scale_a, scale_sh.at[pl.ds(a0, pa)])
        with jax.named_scope("idx_wait"):
            idx_cp.wait()
        with jax.named_scope("barrier"):
            plsc.subcore_barrier()

        n_chunks = B // _CHUNK
        n_writes = 0
        for rr in range(rows_per_w):
            r = wid * rows_per_w + rr
            with jax.named_scope("row_stream_wait"):
                row_cp.wait()

            def pair_body(k2, carry):
                for b, ob in ((0, out_c0), (1, out_c1)):
                    k = k2 * 2 + b
                    pltpu.sync_copy(scale_sh.at[pl.ds(k * _CHUNK, _CHUNK)],
                                    scale_c)

                    @pl.when(carry + b >= 2)
                    def _():
                        pltpu.make_async_copy(
                            out_hbm.at[r, pl.ds(0, _CHUNK)], ob, osem).wait()

                    @plsc.parallel_loop(0, _CHUNK // _LANES, unroll=16)
                    def gbody(v):
                        sl = pl.ds(v * _LANES, _LANES)
                        iv = idx_v[pl.ds(k * _CHUNK + v * _LANES, _LANES)]
                        ob[sl] = plsc.load_gather(row_v, [iv]) * scale_c[sl]
                    pltpu.async_copy(
                        ob, out_hbm.at[r, pl.ds(k * _CHUNK, _CHUNK)], osem)
                return carry + 2
            with jax.named_scope("gather_row"):
                n_writes = lax.fori_loop(0, n_chunks // 2, pair_body,
                                         jnp.int32(n_writes))
            if rr + 1 < rows_per_w:
                row_cp = pltpu.async_copy(embt_hbm.at[r + 1], row_v, rsem)
        for ob in (out_c0, out_c1):
            pltpu.make_async_copy(out_hbm.at[0, pl.ds(0, _CHUNK)],
                                  ob, osem).wait()

    return _k(drugs, doses, emb_t, beta, bias).T
